# baseline (speedup 1.0000x reference)
import jax
import jax.numpy as jnp
from jax import lax
from jax.experimental import pallas as pl
from jax.experimental.pallas import tpu as pltpu

_BH = 16
_VMEM_LIM = 110 * 1024 * 1024


def _cparams():
    return pltpu.CompilerParams(dimension_semantics=("parallel",),
                                vmem_limit_bytes=_VMEM_LIM)


def _full_spec(shape):
    n = len(shape)
    return pl.BlockSpec(shape, lambda i, _n=n: (0,) * _n)


def _w2_taps(w2):
    return w2.reshape(w2.shape[0], 128, 9 * 128)


def _conv1_w_rows(w):
    wt = w[:588].reshape(12, 49, 64).transpose(1, 0, 2)
    wt = jnp.pad(wt, ((0, 0), (0, 4), (0, 0)))
    return wt.reshape(7, 112, 64)


def _run_stack(x, cfgs, layer_blocks, conv1=None, head=None, pv=None,
               ncores=2, manual_w=False):
    bh = 32 // ncores
    mkeys = ("w1", "w3", "wd") if manual_w else ()
    with_conv1 = conv1 is not None
    with_head = head is not None

    args, in_specs = [], []
    if with_conv1:
        xin, wr, tc = conv1
        args += [xin, wr, tc]
        in_specs += [pl.BlockSpec((bh, 12, 32, 32), lambda i: (i, 0, 0, 0)),
                     _full_spec((7, 112, 64)), _full_spec((1, 1024))]
    else:
        H0, W0, C0 = cfgs[0][0], cfgs[0][1], cfgs[0][2]
        args.append(x)
        in_specs.append(pl.BlockSpec((bh, H0, W0, C0),
                                     lambda i: (i, 0, 0, 0)))
    for blocks in layer_blocks:
        for b, blk in enumerate(blocks):
            keys = ["w1", "t1", "w2", "t2", "w3", "t3"]
            if b == 0:
                keys += ["wd", "td"]
            for k in keys:
                a = blk[k]
                args.append(a)
                if k == "w2" or k in mkeys:
                    in_specs.append(pl.BlockSpec(memory_space=pl.ANY))
                else:
                    in_specs.append(_full_spec(a.shape))
    if with_head:
        args.append(pv)
        in_specs.append(pl.BlockSpec((bh, 12), lambda i: (i, 0)))
        for a in head:
            args.append(a)
            in_specs.append(_full_spec(a.shape))
        out_shape = jax.ShapeDtypeStruct((32, 48), jnp.float32)
        out_spec = pl.BlockSpec((bh, 48), lambda i: (i, 0))
    else:
        Hl, Wl = cfgs[-1][0] // cfgs[-1][5], cfgs[-1][1] // cfgs[-1][5]
        Cl = cfgs[-1][4]
        out_shape = jax.ShapeDtypeStruct((32, Hl, Wl, Cl), jnp.bfloat16)
        out_spec = pl.BlockSpec((bh, Hl, Wl, Cl), lambda i: (i, 0, 0, 0))

    scratch = []
    if with_conv1:
        scratch += [pltpu.VMEM((7 * 640, 1024), jnp.bfloat16),
                    pltpu.VMEM((bh, 18, 18, 64), jnp.float32)]
    for (Hin, Win, Cin, width, Cout, s0, nblocks) in cfgs:
        nblk = width // 128
        Ho, Wo = Hin // s0, Win // s0
        Mh = bh * Ho * Wo
        center_rest = (Ho == 1)
        if s0 == 2:
            scratch.append(pltpu.VMEM((bh, Hin + 2, Win + 2, nblk, 128),
                                      jnp.float32))
        else:
            scratch.append(pltpu.VMEM((bh, Hin + 2, Win + 2, width),
                                      jnp.bfloat16))
        if nblocks > 1 and not center_rest:
            scratch.append(pltpu.VMEM((bh, Ho + 2, Wo + 2, width),
                                      jnp.bfloat16))
        else:
            scratch.append(pltpu.VMEM((8, 128), jnp.bfloat16))
        scratch.append(pltpu.VMEM((Mh, nblk * 1152), jnp.bfloat16))
        for b in range(nblocks):
            ntap = 1 if (b > 0 and center_rest) else 9
            scratch.append(pltpu.VMEM((nblk, ntap, 128, 128), jnp.bfloat16))
            if manual_w:
                Ci = Cin if b == 0 else Cout
                scratch.append(pltpu.VMEM((Ci, width), jnp.bfloat16))
                scratch.append(pltpu.VMEM((width, Cout), jnp.bfloat16))
                if b == 0:
                    scratch.append(pltpu.VMEM((Cin, Cout), jnp.bfloat16))
        scratch.append(pltpu.SemaphoreType.DMA((nblocks, 12)))

    def body(*refs):
        it = iter(refs)
        if with_conv1:
            x_ref, wr_ref, tc_ref = next(it), next(it), next(it)
        else:
            x_ref = next(it)
        lref = []
        for (blocks, cfg) in zip(layer_blocks, cfgs):
            blks = []
            for b in range(cfg[6]):
                names = ["w1", "t1", "w2", "t2", "w3", "t3"]
                if b == 0:
                    names += ["wd", "td"]
                blks.append({k: next(it) for k in names})
            lref.append(blks)
        if with_head:
            pv_ref, h1a, h1b, hb1, h2w, hb2, h3w, hb3 = (next(it)
                                                         for _ in range(8))
        out_ref = next(it)
        if with_conv1:
            wb_ref, cpad_ref = next(it), next(it)
        lscr = []
        for (Hin, Win, Cin, width, Cout, s0, nblocks) in cfgs:
            s = {"pad0": next(it), "pad1": next(it), "ps": next(it),
                 "w2": [], "w1": [], "w3": [], "wd": []}
            for b in range(nblocks):
                s["w2"].append(next(it))
                if manual_w:
                    s["w1"].append(next(it))
                    s["w3"].append(next(it))
                    s["wd"].append(next(it) if b == 0 else None)
            s["sem"] = next(it)
            lscr.append(s)

        def _w_copy(li, b, k):
            slot = {"w1": 9, "w3": 10, "wd": 11}[k]
            return pltpu.make_async_copy(
                lref[li][b][k], lscr[li][k][b], lscr[li]["sem"].at[b, slot])

        def _w2_copy(li, b, t, center):
            if center:
                return pltpu.make_async_copy(
                    lref[li][b]["w2"].at[:, :, pl.ds(4 * 128, 128)],
                    lscr[li]["w2"][b].at[:, 0], lscr[li]["sem"].at[b, 0])
            return pltpu.make_async_copy(
                lref[li][b]["w2"].at[:, :, pl.ds(t * 128, 128)],
                lscr[li]["w2"][b].at[:, t], lscr[li]["sem"].at[b, t])

        i = pl.program_id(0)

        for li, cfg in enumerate(cfgs):
            center_rest = (cfg[0] // cfg[5] == 1)
            for b in range(cfg[6]):
                if manual_w:
                    _w_copy(li, b, "w1").start()
                ntap = 1 if (b > 0 and center_rest) else 9
                for t in range(ntap):
                    _w2_copy(li, b, t, ntap == 1).start()
                if manual_w:
                    _w_copy(li, b, "w3").start()
                    if b == 0:
                        _w_copy(li, b, "wd").start()

        if with_conv1:
            wb_ref[...] = jnp.zeros(wb_ref.shape, wb_ref.dtype)
            for ki in range(7):
                wrow = wr_ref[ki]
                for wo in range(16):
                    wb_ref[pl.ds(ki * 640 + 32 * wo, 112),
                           pl.ds(wo * 64, 64)] = wrow
            v = jnp.transpose(x_ref[...], (0, 2, 3, 1))
            v = jnp.pad(v, ((0, 0), (3, 3), (3, 5), (0, 4)))
            v = v.reshape(bh, 19, 2, 640)
            parts = []
            for ki in range(7):
                a, p = divmod(ki, 2)
                parts.append(v[:, a:a + 16, p, :])
            xa = jnp.concatenate(parts, axis=-1)
            xa = xa.reshape(bh * 16, 7 * 640).astype(jnp.bfloat16)
            y = jnp.dot(xa, wb_ref[...],
                        preferred_element_type=jnp.float32) + tc_ref[...]
            y = jnp.maximum(y, 0.0)
            cpad_ref[...] = jnp.zeros(cpad_ref.shape, cpad_ref.dtype)
            cpad_ref[:, 1:17, 1:17, :] = y.reshape(bh, 16, 16, 64)
            m = None
            for di in range(3):
                for dj in range(3):
                    v = cpad_ref[:, pl.Slice(di, 8, 2), pl.Slice(dj, 8, 2), :]
                    m = v if m is None else jnp.maximum(m, v)
            x2d = m.astype(jnp.bfloat16).reshape(bh * 64, 64)
        else:
            x2d = None

        for li, (Hin, Win, Cin, width, Cout, s0, nblocks) in enumerate(cfgs):
            nblk = width // 128
            Ho, Wo = Hin // s0, Win // s0
            Mh = bh * Ho * Wo
            center_rest = (Ho == 1)
            pad0_ref = lscr[li]["pad0"]
            pad1_ref = lscr[li]["pad1"]
            ps_ref = lscr[li]["ps"]
            w2scr = lscr[li]["w2"]

            for b in range(nblocks):
                r = lref[li][b]
                if b == 0:
                    Hb, Wb = Hin, Win
                    if x2d is None:
                        xf = x_ref[...].reshape(bh * Hin * Win, Cin)
                    else:
                        xf = x2d
                else:
                    Hb, Wb = Ho, Wo
                    xf = x2d

                if manual_w:
                    _w_copy(li, b, "w1").wait()
                    w1v = lscr[li]["w1"][b][...]
                else:
                    w1v = r["w1"][...]
                h1 = jnp.dot(xf, w1v,
                             preferred_element_type=jnp.float32) + r["t1"][...]
                h1 = jnp.maximum(h1, 0.0).astype(jnp.bfloat16)

                xs = None
                if b > 0 and center_rest:
                    _w2_copy(li, b, 0, True).wait()
                    parts = []
                    for j in range(nblk):
                        wc = w2scr[b][j, 0]
                        parts.append(jnp.dot(
                            h1[:, j * 128:(j + 1) * 128], wc,
                            preferred_element_type=jnp.float32))
                    h2 = jnp.concatenate(parts, axis=-1) + r["t2"][...]
                else:
                    pad_ref = pad0_ref if b == 0 else pad1_ref
                    if b <= 1:
                        pad_ref[...] = jnp.zeros(pad_ref.shape, pad_ref.dtype)
                    if b == 0 and s0 == 2:
                        pad_ref[:, 1:Hb + 1, 1:Wb + 1, :, :] = (
                            xf.astype(jnp.float32)
                            .reshape(bh, Hb, Wb, nblk, 128))
                        xs = pad_ref[:, pl.Slice(1, Ho, 2),
                                     pl.Slice(1, Wo, 2), :, :]
                        xs = xs.reshape(Mh, Cin).astype(jnp.bfloat16)
                        pad_ref[:, 1:Hb + 1, 1:Wb + 1, :, :] = (
                            h1.reshape(bh, Hb, Wb, nblk, 128)
                            .astype(pad_ref.dtype))
                        for t in range(9):
                            ki, kj = divmod(t, 3)
                            xt = pad_ref[:, pl.Slice(ki, Ho, 2),
                                         pl.Slice(kj, Wo, 2), :, :]
                            xt = xt.reshape(Mh, width).astype(jnp.bfloat16)
                            for j in range(nblk):
                                ps_ref[:, (j * 9 + t) * 128:
                                       (j * 9 + t + 1) * 128] = (
                                    xt[:, j * 128:(j + 1) * 128])
                    else:
                        pad_ref[:, 1:Hb + 1, 1:Wb + 1, :] = (
                            h1.reshape(bh, Hb, Wb, width))
                        for t in range(9):
                            ki, kj = divmod(t, 3)
                            xt = pad_ref[:, pl.Slice(ki, Ho, 1),
                                         pl.Slice(kj, Wo, 1), :]
                            xt = xt.reshape(Mh, width)
                            for j in range(nblk):
                                ps_ref[:, (j * 9 + t) * 128:
                                       (j * 9 + t + 1) * 128] = (
                                    xt[:, j * 128:(j + 1) * 128])
                    for t in range(9):
                        _w2_copy(li, b, t, False).wait()
                    parts = []
                    for j in range(nblk):
                        wj = w2scr[b][j].reshape(1152, 128)
                        parts.append(jnp.dot(
                            ps_ref[:, j * 1152:(j + 1) * 1152], wj,
                            preferred_element_type=jnp.float32))
                    h2 = jnp.concatenate(parts, axis=-1) + r["t2"][...]
                h2 = jnp.maximum(h2, 0.0).astype(jnp.bfloat16)

                if manual_w:
                    _w_copy(li, b, "w3").wait()
                    w3v = lscr[li]["w3"][b][...]
                else:
                    w3v = r["w3"][...]
                y = jnp.dot(h2, w3v,
                            preferred_element_type=jnp.float32) + r["t3"][...]
                if b == 0:
                    if s0 != 2:
                        xs = xf
                    if manual_w:
                        _w_copy(li, b, "wd").wait()
                        wdv = lscr[li]["wd"][b][...]
                    else:
                        wdv = r["wd"][...]
                    y = y + jnp.dot(xs, wdv,
                                    preferred_element_type=jnp.float32
                                    ) + r["td"][...]
                else:
                    y = y + x2d.astype(jnp.float32)
                x2d = jnp.maximum(y, 0.0).astype(jnp.bfloat16)

        if with_head:
            h = (jnp.dot(x2d, h1a[...], preferred_element_type=jnp.float32)
                 + jnp.dot(pv_ref[...], h1b[...],
                           preferred_element_type=jnp.float32) + hb1[...])
            h = jnp.where(h > 0.0, h, 0.1 * h).astype(jnp.bfloat16)
            h = jnp.dot(h, h2w[...],
                        preferred_element_type=jnp.float32) + hb2[...]
            h = jnp.where(h > 0.0, h, 0.1 * h).astype(jnp.bfloat16)
            h = jnp.dot(h, h3w[...],
                        preferred_element_type=jnp.float32) + hb3[...]
            out_ref[...] = jax.nn.sigmoid(h)
        else:
            Hl, Wl = cfgs[-1][0] // cfgs[-1][5], cfgs[-1][1] // cfgs[-1][5]
            out_ref[...] = x2d.reshape(bh, Hl, Wl, cfgs[-1][4])

    return pl.pallas_call(
        body,
        out_shape=out_shape,
        grid=(ncores,),
        in_specs=in_specs,
        out_specs=out_spec,
        scratch_shapes=scratch,
        compiler_params=_cparams(),
    )(*args)


_L_CFG = [
    (8, 8, 64, 128, 256, 1, 3),
    (8, 8, 256, 256, 512, 2, 4),
    (4, 4, 512, 512, 1024, 2, 6),
    (2, 2, 1024, 1024, 2048, 2, 3),
]


def _prep_blocks(layer_ws, li):
    width = _L_CFG[li][3]
    Cout = _L_CFG[li][4]
    blocks = []
    for b, p in enumerate(layer_ws[li]):
        blk = {
            "w1": p["w1"],
            "t1": p["t1"].reshape(1, width).astype(jnp.float32),
            "w2": _w2_taps(p["w2"]),
            "t2": p["t2"].reshape(1, width).astype(jnp.float32),
            "w3": p["w3"],
            "t3": p["t3"].reshape(1, Cout).astype(jnp.float32),
        }
        if b == 0:
            blk["wd"] = p["wd"]
            blk["td"] = p["td"].reshape(1, Cout).astype(jnp.float32)
        blocks.append(blk)
    return blocks


@jax.jit
def _forward(x_nchw, pv, bn1_t, conv1_w, layer_ws, head_ws):
    t1024 = jnp.tile(bn1_t.astype(jnp.float32), 16).reshape(1, 1024)

    h1_w, h1_b, h2_w, h2_b, h3_w, h3_b = head_ws
    mid = _run_stack(None, _L_CFG[:2],
                     [_prep_blocks(layer_ws, 0), _prep_blocks(layer_ws, 1)],
                     manual_w=True,
                     conv1=(x_nchw, _conv1_w_rows(conv1_w), t1024))
    return _run_stack(mid, _L_CFG[2:],
                      [_prep_blocks(layer_ws, 2), _prep_blocks(layer_ws, 3)],
                      ncores=1, manual_w=True,
                      pv=pv.astype(jnp.bfloat16),
                      head=[h1_w[:2048], h1_w[2048:],
                            h1_b.reshape(1, 256).astype(jnp.float32),
                            h2_w, h2_b.reshape(1, 256).astype(jnp.float32),
                            h3_w, h3_b.reshape(1, 48).astype(jnp.float32)])


def kernel(hrv, pv, bn1_t, conv1_w, L0b0_t1, L0b0_w1, L0b0_t2, L0b0_w2, L0b0_t3, L0b0_w3, L0b0_td, L0b0_wd, L0b1_t1, L0b1_w1, L0b1_t2, L0b1_w2, L0b1_t3, L0b1_w3, L0b2_t1, L0b2_w1, L0b2_t2, L0b2_w2, L0b2_t3, L0b2_w3, L1b0_t1, L1b0_w1, L1b0_t2, L1b0_w2, L1b0_t3, L1b0_w3, L1b0_td, L1b0_wd, L1b1_t1, L1b1_w1, L1b1_t2, L1b1_w2, L1b1_t3, L1b1_w3, L1b2_t1, L1b2_w1, L1b2_t2, L1b2_w2, L1b2_t3, L1b2_w3, L1b3_t1, L1b3_w1, L1b3_t2, L1b3_w2, L1b3_t3, L1b3_w3, L2b0_t1, L2b0_w1, L2b0_t2, L2b0_w2, L2b0_t3, L2b0_w3, L2b0_td, L2b0_wd, L2b1_t1, L2b1_w1, L2b1_t2, L2b1_w2, L2b1_t3, L2b1_w3, L2b2_t1, L2b2_w1, L2b2_t2, L2b2_w2, L2b2_t3, L2b2_w3, L2b3_t1, L2b3_w1, L2b3_t2, L2b3_w2, L2b3_t3, L2b3_w3, L2b4_t1, L2b4_w1, L2b4_t2, L2b4_w2, L2b4_t3, L2b4_w3, L2b5_t1, L2b5_w1, L2b5_t2, L2b5_w2, L2b5_t3, L2b5_w3, L3b0_t1, L3b0_w1, L3b0_t2, L3b0_w2, L3b0_t3, L3b0_w3, L3b0_td, L3b0_wd, L3b1_t1, L3b1_w1, L3b1_t2, L3b1_w2, L3b1_t3, L3b1_w3, L3b2_t1, L3b2_w1, L3b2_t2, L3b2_w2, L3b2_t3, L3b2_w3, h1_w, h1_b, h2_w, h2_b, h3_w, h3_b):
    _a = dict(locals())
    layer_ws = []
    for li, nb in enumerate([3, 4, 6, 3]):
        blocks = []
        for b in range(nb):
            p = {k: _a["L%db%d_%s" % (li, b, k)]
                 for k in ("t1", "w1", "t2", "w2", "t3", "w3")}
            if b == 0:
                p["td"] = _a["L%db%d_td" % (li, b)]
                p["wd"] = _a["L%db%d_wd" % (li, b)]
            blocks.append(p)
        layer_ws.append(blocks)
    return _forward(hrv, pv, bn1_t, conv1_w, layer_ws,
                    (h1_w, h1_b, h2_w, h2_b, h3_w, h3_b))


# speedup vs baseline: 1.0309x; 1.0309x over previous
import jax
import jax.numpy as jnp
from jax import lax
from jax.experimental import pallas as pl
from jax.experimental.pallas import tpu as pltpu

_BH = 16
_VMEM_LIM = 110 * 1024 * 1024


def _cparams():
    return pltpu.CompilerParams(dimension_semantics=("parallel",),
                                vmem_limit_bytes=_VMEM_LIM)


def _full_spec(shape):
    n = len(shape)
    return pl.BlockSpec(shape, lambda i, _n=n: (0,) * _n)


def _w2_taps(w2):
    return w2.reshape(w2.shape[0], 128, 9 * 128)


def _conv1_w_rows(w):
    wt = w[:588].reshape(12, 49, 64).transpose(1, 0, 2)
    wt = jnp.pad(wt, ((0, 0), (0, 4), (0, 0)))
    return wt.reshape(7, 112, 64)


def _run_stack(x, cfgs, layer_blocks, conv1=None, head=None, pv=None,
               ncores=2, manual_w=False):
    bh = 32 // ncores
    mkeys = ("w1", "w3", "wd") if manual_w else ()
    with_conv1 = conv1 is not None
    with_head = head is not None

    args, in_specs = [], []
    if with_conv1:
        xin, wr, tc = conv1
        args += [xin, wr, tc]
        in_specs += [pl.BlockSpec((bh, 12, 32, 32), lambda i: (i, 0, 0, 0)),
                     _full_spec((7, 112, 64)), _full_spec((1, 1024))]
    else:
        H0, W0, C0 = cfgs[0][0], cfgs[0][1], cfgs[0][2]
        args.append(x)
        in_specs.append(pl.BlockSpec((bh, H0, W0, C0),
                                     lambda i: (i, 0, 0, 0)))
    for blocks in layer_blocks:
        for b, blk in enumerate(blocks):
            keys = ["w1", "t1", "w2", "t2", "w3", "t3"]
            if b == 0:
                keys += ["wd", "td"]
            for k in keys:
                a = blk[k]
                args.append(a)
                if k == "w2" or k in mkeys:
                    in_specs.append(pl.BlockSpec(memory_space=pl.ANY))
                else:
                    in_specs.append(_full_spec(a.shape))
    if with_head:
        args.append(pv)
        in_specs.append(pl.BlockSpec((bh, 12), lambda i: (i, 0)))
        for a in head:
            args.append(a)
            in_specs.append(_full_spec(a.shape))
        out_shape = jax.ShapeDtypeStruct((32, 48), jnp.float32)
        out_spec = pl.BlockSpec((bh, 48), lambda i: (i, 0))
    else:
        Hl, Wl = cfgs[-1][0] // cfgs[-1][5], cfgs[-1][1] // cfgs[-1][5]
        Cl = cfgs[-1][4]
        out_shape = jax.ShapeDtypeStruct((32, Hl, Wl, Cl), jnp.bfloat16)
        out_spec = pl.BlockSpec((bh, Hl, Wl, Cl), lambda i: (i, 0, 0, 0))

    scratch = []
    if with_conv1:
        scratch += [pltpu.VMEM((7 * 640, 1024), jnp.bfloat16),
                    pltpu.VMEM((bh, 18, 18, 64), jnp.float32)]
    for (Hin, Win, Cin, width, Cout, s0, nblocks) in cfgs:
        nblk = width // 128
        Ho, Wo = Hin // s0, Win // s0
        Mh = bh * Ho * Wo
        center_rest = (Ho == 1)
        if s0 == 2:
            scratch.append(pltpu.VMEM((bh, Hin + 2, Win + 2, nblk, 128),
                                      jnp.float32))
        else:
            scratch.append(pltpu.VMEM((bh, Hin + 2, Win + 2, width),
                                      jnp.bfloat16))
        if nblocks > 1 and not center_rest:
            scratch.append(pltpu.VMEM((bh, Ho + 2, Wo + 2, width),
                                      jnp.bfloat16))
        else:
            scratch.append(pltpu.VMEM((8, 128), jnp.bfloat16))
        scratch.append(pltpu.VMEM((Mh, nblk * 1152), jnp.bfloat16))
        for b in range(nblocks):
            ntap = 1 if (b > 0 and center_rest) else 9
            scratch.append(pltpu.VMEM((nblk, ntap, 128, 128), jnp.bfloat16))
            if manual_w:
                Ci = Cin if b == 0 else Cout
                scratch.append(pltpu.VMEM((Ci, width), jnp.bfloat16))
                scratch.append(pltpu.VMEM((width, Cout), jnp.bfloat16))
                if b == 0:
                    scratch.append(pltpu.VMEM((Cin, Cout), jnp.bfloat16))
        scratch.append(pltpu.SemaphoreType.DMA((nblocks, 12)))

    def body(*refs):
        it = iter(refs)
        if with_conv1:
            x_ref, wr_ref, tc_ref = next(it), next(it), next(it)
        else:
            x_ref = next(it)
        lref = []
        for (blocks, cfg) in zip(layer_blocks, cfgs):
            blks = []
            for b in range(cfg[6]):
                names = ["w1", "t1", "w2", "t2", "w3", "t3"]
                if b == 0:
                    names += ["wd", "td"]
                blks.append({k: next(it) for k in names})
            lref.append(blks)
        if with_head:
            pv_ref, h1a, h1b, hb1, h2w, hb2, h3w, hb3 = (next(it)
                                                         for _ in range(8))
        out_ref = next(it)
        if with_conv1:
            wb_ref, cpad_ref = next(it), next(it)
        lscr = []
        for (Hin, Win, Cin, width, Cout, s0, nblocks) in cfgs:
            s = {"pad0": next(it), "pad1": next(it), "ps": next(it),
                 "w2": [], "w1": [], "w3": [], "wd": []}
            for b in range(nblocks):
                s["w2"].append(next(it))
                if manual_w:
                    s["w1"].append(next(it))
                    s["w3"].append(next(it))
                    s["wd"].append(next(it) if b == 0 else None)
            s["sem"] = next(it)
            lscr.append(s)

        def _w_copy(li, b, k):
            slot = {"w1": 9, "w3": 10, "wd": 11}[k]
            return pltpu.make_async_copy(
                lref[li][b][k], lscr[li][k][b], lscr[li]["sem"].at[b, slot])

        def _w2_copy(li, b, t, center):
            if center:
                return pltpu.make_async_copy(
                    lref[li][b]["w2"].at[:, :, pl.ds(4 * 128, 128)],
                    lscr[li]["w2"][b].at[:, 0], lscr[li]["sem"].at[b, 0])
            return pltpu.make_async_copy(
                lref[li][b]["w2"].at[:, :, pl.ds(t * 128, 128)],
                lscr[li]["w2"][b].at[:, t], lscr[li]["sem"].at[b, t])

        i = pl.program_id(0)

        for li, cfg in enumerate(cfgs):
            center_rest = (cfg[0] // cfg[5] == 1)
            for b in range(cfg[6]):
                if manual_w:
                    _w_copy(li, b, "w1").start()
                ntap = 1 if (b > 0 and center_rest) else 9
                for t in range(ntap):
                    _w2_copy(li, b, t, ntap == 1).start()
                if manual_w:
                    _w_copy(li, b, "w3").start()
                    if b == 0:
                        _w_copy(li, b, "wd").start()

        if with_conv1:
            wb_ref[...] = jnp.zeros(wb_ref.shape, wb_ref.dtype)
            for ki in range(7):
                wrow = wr_ref[ki]
                for wo in range(16):
                    wb_ref[pl.ds(ki * 640 + 32 * wo, 112),
                           pl.ds(wo * 64, 64)] = wrow
            v = jnp.transpose(x_ref[...], (0, 2, 3, 1))
            v = jnp.pad(v, ((0, 0), (3, 3), (3, 5), (0, 4)))
            v = v.reshape(bh, 19, 2, 640)
            parts = []
            for ki in range(7):
                a, p = divmod(ki, 2)
                parts.append(v[:, a:a + 16, p, :])
            xa = jnp.concatenate(parts, axis=-1)
            xa = xa.reshape(bh * 16, 7 * 640).astype(jnp.bfloat16)
            y = jnp.dot(xa, wb_ref[...],
                        preferred_element_type=jnp.float32) + tc_ref[...]
            y = jnp.maximum(y, 0.0)
            cpad_ref[...] = jnp.zeros(cpad_ref.shape, cpad_ref.dtype)
            cpad_ref[:, 1:17, 1:17, :] = y.reshape(bh, 16, 16, 64)
            m = None
            for di in range(3):
                for dj in range(3):
                    v = cpad_ref[:, pl.Slice(di, 8, 2), pl.Slice(dj, 8, 2), :]
                    m = v if m is None else jnp.maximum(m, v)
            x2d = m.astype(jnp.bfloat16).reshape(bh * 64, 64)
        else:
            x2d = None

        for li, (Hin, Win, Cin, width, Cout, s0, nblocks) in enumerate(cfgs):
            nblk = width // 128
            Ho, Wo = Hin // s0, Win // s0
            Mh = bh * Ho * Wo
            center_rest = (Ho == 1)
            pad0_ref = lscr[li]["pad0"]
            pad1_ref = lscr[li]["pad1"]
            ps_ref = lscr[li]["ps"]
            w2scr = lscr[li]["w2"]

            for b in range(nblocks):
                r = lref[li][b]
                if b == 0:
                    Hb, Wb = Hin, Win
                    if x2d is None:
                        xf = x_ref[...].reshape(bh * Hin * Win, Cin)
                    else:
                        xf = x2d
                else:
                    Hb, Wb = Ho, Wo
                    xf = x2d

                if manual_w:
                    _w_copy(li, b, "w1").wait()
                    w1v = lscr[li]["w1"][b][...]
                else:
                    w1v = r["w1"][...]
                h1 = jnp.dot(xf, w1v,
                             preferred_element_type=jnp.float32) + r["t1"][...]
                h1 = jnp.maximum(h1, 0.0).astype(jnp.bfloat16)

                xs = None
                if b > 0 and center_rest:
                    _w2_copy(li, b, 0, True).wait()
                    parts = []
                    for j in range(nblk):
                        wc = w2scr[b][j, 0]
                        parts.append(jnp.dot(
                            h1[:, j * 128:(j + 1) * 128], wc,
                            preferred_element_type=jnp.float32))
                    h2 = jnp.concatenate(parts, axis=-1) + r["t2"][...]
                else:
                    pad_ref = pad0_ref if b == 0 else pad1_ref
                    if b <= 1:
                        pad_ref[...] = jnp.zeros(pad_ref.shape, pad_ref.dtype)
                    if b == 0 and s0 == 2:
                        pad_ref[:, 1:Hb + 1, 1:Wb + 1, :, :] = (
                            xf.astype(jnp.float32)
                            .reshape(bh, Hb, Wb, nblk, 128))
                        xs = pad_ref[:, pl.Slice(1, Ho, 2),
                                     pl.Slice(1, Wo, 2), :, :]
                        xs = xs.reshape(Mh, Cin).astype(jnp.bfloat16)
                        pad_ref[:, 1:Hb + 1, 1:Wb + 1, :, :] = (
                            h1.reshape(bh, Hb, Wb, nblk, 128)
                            .astype(pad_ref.dtype))
                        for t in range(9):
                            ki, kj = divmod(t, 3)
                            xt = pad_ref[:, pl.Slice(ki, Ho, 2),
                                         pl.Slice(kj, Wo, 2), :, :]
                            xt = xt.reshape(Mh, width).astype(jnp.bfloat16)
                            for j in range(nblk):
                                ps_ref[:, (j * 9 + t) * 128:
                                       (j * 9 + t + 1) * 128] = (
                                    xt[:, j * 128:(j + 1) * 128])
                    else:
                        pad_ref[:, 1:Hb + 1, 1:Wb + 1, :] = (
                            h1.reshape(bh, Hb, Wb, width))
                        for t in range(9):
                            ki, kj = divmod(t, 3)
                            xt = pad_ref[:, pl.Slice(ki, Ho, 1),
                                         pl.Slice(kj, Wo, 1), :]
                            xt = xt.reshape(Mh, width)
                            for j in range(nblk):
                                ps_ref[:, (j * 9 + t) * 128:
                                       (j * 9 + t + 1) * 128] = (
                                    xt[:, j * 128:(j + 1) * 128])
                    for t in range(9):
                        _w2_copy(li, b, t, False).wait()
                    parts = []
                    for j in range(nblk):
                        wj = w2scr[b][j].reshape(1152, 128)
                        parts.append(jnp.dot(
                            ps_ref[:, j * 1152:(j + 1) * 1152], wj,
                            preferred_element_type=jnp.float32))
                    h2 = jnp.concatenate(parts, axis=-1) + r["t2"][...]
                h2 = jnp.maximum(h2, 0.0).astype(jnp.bfloat16)

                if manual_w:
                    _w_copy(li, b, "w3").wait()
                    w3v = lscr[li]["w3"][b][...]
                else:
                    w3v = r["w3"][...]
                y = jnp.dot(h2, w3v,
                            preferred_element_type=jnp.float32) + r["t3"][...]
                if b == 0:
                    if s0 != 2:
                        xs = xf
                    if manual_w:
                        _w_copy(li, b, "wd").wait()
                        wdv = lscr[li]["wd"][b][...]
                    else:
                        wdv = r["wd"][...]
                    y = y + jnp.dot(xs, wdv,
                                    preferred_element_type=jnp.float32
                                    ) + r["td"][...]
                else:
                    y = y + x2d.astype(jnp.float32)
                x2d = jnp.maximum(y, 0.0).astype(jnp.bfloat16)

        if with_head:
            h = (jnp.dot(x2d, h1a[...], preferred_element_type=jnp.float32)
                 + jnp.dot(pv_ref[...], h1b[...],
                           preferred_element_type=jnp.float32) + hb1[...])
            h = jnp.where(h > 0.0, h, 0.1 * h).astype(jnp.bfloat16)
            h = jnp.dot(h, h2w[...],
                        preferred_element_type=jnp.float32) + hb2[...]
            h = jnp.where(h > 0.0, h, 0.1 * h).astype(jnp.bfloat16)
            h = jnp.dot(h, h3w[...],
                        preferred_element_type=jnp.float32) + hb3[...]
            out_ref[...] = jax.nn.sigmoid(h)
        else:
            Hl, Wl = cfgs[-1][0] // cfgs[-1][5], cfgs[-1][1] // cfgs[-1][5]
            out_ref[...] = x2d.reshape(bh, Hl, Wl, cfgs[-1][4])

    return pl.pallas_call(
        body,
        out_shape=out_shape,
        grid=(ncores,),
        in_specs=in_specs,
        out_specs=out_spec,
        scratch_shapes=scratch,
        compiler_params=_cparams(),
    )(*args)


_L_CFG = [
    (8, 8, 64, 128, 256, 1, 3),
    (8, 8, 256, 256, 512, 2, 4),
    (4, 4, 512, 512, 1024, 2, 6),
    (2, 2, 1024, 1024, 2048, 2, 3),
]


def _prep_blocks(layer_ws, li):
    width = _L_CFG[li][3]
    Cout = _L_CFG[li][4]
    blocks = []
    for b, p in enumerate(layer_ws[li]):
        blk = {
            "w1": p["w1"],
            "t1": p["t1"].reshape(1, width).astype(jnp.float32),
            "w2": _w2_taps(p["w2"]),
            "t2": p["t2"].reshape(1, width).astype(jnp.float32),
            "w3": p["w3"],
            "t3": p["t3"].reshape(1, Cout).astype(jnp.float32),
        }
        if b == 0:
            blk["wd"] = p["wd"]
            blk["td"] = p["td"].reshape(1, Cout).astype(jnp.float32)
        blocks.append(blk)
    return blocks


@jax.jit
def _forward(x_nchw, pv, bn1_t, conv1_w, layer_ws, head_ws):
    t1024 = jnp.tile(bn1_t.astype(jnp.float32), 16).reshape(1, 1024)

    h1_w, h1_b, h2_w, h2_b, h3_w, h3_b = head_ws
    mid = _run_stack(None, _L_CFG[:2],
                     [_prep_blocks(layer_ws, 0), _prep_blocks(layer_ws, 1)],
                     conv1=(x_nchw, _conv1_w_rows(conv1_w), t1024))
    return _run_stack(mid, _L_CFG[2:],
                      [_prep_blocks(layer_ws, 2), _prep_blocks(layer_ws, 3)],
                      ncores=1, manual_w=True,
                      pv=pv.astype(jnp.bfloat16),
                      head=[h1_w[:2048], h1_w[2048:],
                            h1_b.reshape(1, 256).astype(jnp.float32),
                            h2_w, h2_b.reshape(1, 256).astype(jnp.float32),
                            h3_w, h3_b.reshape(1, 48).astype(jnp.float32)])


def kernel(hrv, pv, bn1_t, conv1_w, L0b0_t1, L0b0_w1, L0b0_t2, L0b0_w2, L0b0_t3, L0b0_w3, L0b0_td, L0b0_wd, L0b1_t1, L0b1_w1, L0b1_t2, L0b1_w2, L0b1_t3, L0b1_w3, L0b2_t1, L0b2_w1, L0b2_t2, L0b2_w2, L0b2_t3, L0b2_w3, L1b0_t1, L1b0_w1, L1b0_t2, L1b0_w2, L1b0_t3, L1b0_w3, L1b0_td, L1b0_wd, L1b1_t1, L1b1_w1, L1b1_t2, L1b1_w2, L1b1_t3, L1b1_w3, L1b2_t1, L1b2_w1, L1b2_t2, L1b2_w2, L1b2_t3, L1b2_w3, L1b3_t1, L1b3_w1, L1b3_t2, L1b3_w2, L1b3_t3, L1b3_w3, L2b0_t1, L2b0_w1, L2b0_t2, L2b0_w2, L2b0_t3, L2b0_w3, L2b0_td, L2b0_wd, L2b1_t1, L2b1_w1, L2b1_t2, L2b1_w2, L2b1_t3, L2b1_w3, L2b2_t1, L2b2_w1, L2b2_t2, L2b2_w2, L2b2_t3, L2b2_w3, L2b3_t1, L2b3_w1, L2b3_t2, L2b3_w2, L2b3_t3, L2b3_w3, L2b4_t1, L2b4_w1, L2b4_t2, L2b4_w2, L2b4_t3, L2b4_w3, L2b5_t1, L2b5_w1, L2b5_t2, L2b5_w2, L2b5_t3, L2b5_w3, L3b0_t1, L3b0_w1, L3b0_t2, L3b0_w2, L3b0_t3, L3b0_w3, L3b0_td, L3b0_wd, L3b1_t1, L3b1_w1, L3b1_t2, L3b1_w2, L3b1_t3, L3b1_w3, L3b2_t1, L3b2_w1, L3b2_t2, L3b2_w2, L3b2_t3, L3b2_w3, h1_w, h1_b, h2_w, h2_b, h3_w, h3_b):
    _a = dict(locals())
    layer_ws = []
    for li, nb in enumerate([3, 4, 6, 3]):
        blocks = []
        for b in range(nb):
            p = {k: _a["L%db%d_%s" % (li, b, k)]
                 for k in ("t1", "w1", "t2", "w2", "t3", "w3")}
            if b == 0:
                p["td"] = _a["L%db%d_td" % (li, b)]
                p["wd"] = _a["L%db%d_wd" % (li, b)]
            blocks.append(p)
        layer_ws.append(blocks)
    return _forward(hrv, pv, bn1_t, conv1_w, layer_ws,
                    (h1_w, h1_b, h2_w, h2_b, h3_w, h3_b))
